# baseline (speedup 1.0000x reference)
import dataclasses
import functools

import jax
import jax.numpy as jnp
from jax import lax
from jax.experimental import pallas as pl
from jax.experimental.pallas import tpu as pltpu
from jax.experimental.pallas import tpu_sc as plsc

B = 16384
L = 20
EMB = 32
VOCAB = 1000000
PACK = 128 // EMB
N = B * L

NC = 2
NS = 16
NW = NC * NS
LK_PER_W = N // NW
B_PER_W = B // NW
CHUNK = 80
CBATCH = CHUNK // L
N_CHUNKS = LK_PER_W // CHUNK
LANES = 16


def _compiler_params():
    cp = pltpu.CompilerParams()
    if "needs_layout_passes" in pltpu.CompilerParams.__dataclass_fields__:
        cp = dataclasses.replace(cp, needs_layout_passes=False)
    return cp


PK = 320
N_SLABS = VOCAB // PK


def _sc_pack(table):
    mesh = plsc.VectorSubcoreMesh(core_axis_name="c", subcore_axis_name="s")

    @functools.partial(
        pl.kernel,
        mesh=mesh,
        compiler_params=_compiler_params(),
        out_type=jax.ShapeDtypeStruct((VOCAB // PACK, PACK * EMB),
                                      jnp.float32),
        scratch_types=[
            pltpu.VMEM((PK, EMB), jnp.float32),
            pltpu.VMEM((PK, EMB), jnp.float32),
            pltpu.VMEM((PK // PACK, 128), jnp.float32),
            pltpu.VMEM((PK // PACK, 128), jnp.float32),
            pltpu.SemaphoreType.DMA,
            pltpu.SemaphoreType.DMA,
            pltpu.SemaphoreType.DMA,
            pltpu.SemaphoreType.DMA,
        ],
    )
    def k(tbl_hbm, out_hbm, i0, i1, o0, o1, l0, l1, s0, s1):
        wid = lax.axis_index("s") * NC + lax.axis_index("c")
        ins = (i0, i1)
        outs = (o0, o1)
        lsems = (l0, l1)
        ssems = (s0, s1)
        n_s = (N_SLABS - wid + NW - 1) // NW

        def slab(q):
            return q * NW + wid

        def start_load(q, p):
            pltpu.make_async_copy(
                tbl_hbm.at[pl.ds(slab(q) * PK, PK)], ins[p], lsems[p]
            ).start()

        def wait_load(q, p):
            pltpu.make_async_copy(
                tbl_hbm.at[pl.ds(slab(q) * PK, PK)], ins[p], lsems[p]
            ).wait()

        def start_store(q, p):
            pltpu.make_async_copy(
                outs[p], out_hbm.at[pl.ds(slab(q) * (PK // PACK), PK // PACK)],
                ssems[p],
            ).start()

        def wait_store(q, p):
            pltpu.make_async_copy(
                outs[p], out_hbm.at[pl.ds(slab(q) * (PK // PACK), PK // PACK)],
                ssems[p],
            ).wait()

        def shuffle(p):
            i, o = ins[p], outs[p]

            @pl.loop(0, PK, step=PACK)
            def _(r4):
                for kk in range(PACK):
                    for h in range(EMB // LANES):
                        o[r4 // PACK,
                          pl.ds(kk * EMB + h * LANES, LANES)] = (
                            i[r4 + kk, pl.ds(h * LANES, LANES)]
                        )

        @pl.when(n_s > 0)
        def _():
            start_load(0, 0)

        @pl.when(n_s > 1)
        def _():
            start_load(1, 1)

        @pl.loop(0, (N_SLABS // NW + 1) // 2 + 1)
        def _(q2):
            for p in range(2):
                q = q2 * 2 + p

                @pl.when(q < n_s)
                def _():
                    wait_load(q, p)

                    @pl.when(q >= 2)
                    def _():
                        wait_store(q - 2, p)

                    shuffle(p)
                    start_store(q, p)

                    @pl.when(q + 2 < n_s)
                    def _():
                        start_load(q + 2, p)

        for p in range(2):
            @pl.when((n_s >= 2) & ((n_s - 2) % 2 == p))
            def _():
                wait_store(n_s - 2, p)

            @pl.when((n_s >= 1) & ((n_s - 1) % 2 == p))
            def _():
                wait_store(n_s - 1, p)

    return k(table)


def _sc_gather(packed, indices):
    mesh = plsc.VectorSubcoreMesh(core_axis_name="c", subcore_axis_name="s")

    @functools.partial(
        pl.kernel,
        mesh=mesh,
        compiler_params=_compiler_params(),
        out_type=jax.ShapeDtypeStruct((B, L, EMB), jnp.float32),
        scratch_types=[
            pltpu.VMEM((LK_PER_W,), jnp.int32),
            pltpu.VMEM((LK_PER_W,), jnp.int32),
            pltpu.VMEM((CHUNK, 128), jnp.float32),
            pltpu.VMEM((CHUNK, 128), jnp.float32),
            pltpu.VMEM((CHUNK, EMB), jnp.float32),
            pltpu.VMEM((CHUNK, EMB), jnp.float32),
            pltpu.SemaphoreType.DMA,
            pltpu.SemaphoreType.DMA,
            pltpu.SemaphoreType.DMA,
            pltpu.SemaphoreType.DMA,
        ],
    )
    def k(tbl_hbm, idx_hbm, out_hbm, idx_v, pidx_v, b0, b1, o0, o1,
          g0, g1, s0, s1):
        wid = lax.axis_index("s") * NC + lax.axis_index("c")
        lbase = wid * LK_PER_W
        bbase = wid * B_PER_W

        bufs = (b0, b1)
        outs = (o0, o1)
        gsems = (g0, g1)
        ssems = (s0, s1)

        pltpu.sync_copy(idx_hbm.at[pl.ds(lbase, LK_PER_W)], idx_v)

        @pl.loop(0, LK_PER_W, step=LANES)
        def _(i):
            pidx_v[pl.ds(i, LANES)] = idx_v[pl.ds(i, LANES)] >> 2

        def start_gather(c, p):
            pltpu.make_async_copy(
                tbl_hbm.at[pidx_v.at[pl.ds(c * CHUNK, CHUNK)]],
                bufs[p], gsems[p],
            ).start()

        def wait_gather(c, p):
            pltpu.make_async_copy(
                tbl_hbm.at[pidx_v.at[pl.ds(c * CHUNK, CHUNK)]],
                bufs[p], gsems[p],
            ).wait()

        def start_store(c, p):
            for bb in range(CBATCH):
                pltpu.make_async_copy(
                    outs[p].at[pl.ds(bb * L, L)],
                    out_hbm.at[bbase + c * CBATCH + bb],
                    ssems[p],
                ).start()

        def wait_store(c, p):
            for bb in range(CBATCH):
                pltpu.make_async_copy(
                    outs[p].at[pl.ds(bb * L, L)],
                    out_hbm.at[bbase + c * CBATCH + bb],
                    ssems[p],
                ).wait()

        def select(c, p):
            buf, o = bufs[p], outs[p]

            @pl.loop(0, CHUNK, step=LANES)
            def _(t):
                colb = (idx_v[pl.ds(c * CHUNK + t, LANES)] & 3) * EMB
                for j in range(LANES):
                    q = colb[j]
                    r = t + j
                    o[r, pl.ds(0, LANES)] = buf[r, pl.ds(q, LANES)]
                    o[r, pl.ds(LANES, LANES)] = buf[r, pl.ds(q + LANES, LANES)]

        start_gather(0, 0)
        start_gather(1, 1)

        @pl.loop(0, N_CHUNKS // 2)
        def _(c2):
            for p in range(2):
                c = c2 * 2 + p
                wait_gather(c, p)

                @pl.when(c2 > 0)
                def _():
                    wait_store(c - 2, p)

                select(c, p)
                start_store(c, p)

                @pl.when(c + 2 < N_CHUNKS)
                def _():
                    start_gather(c + 2, p)

        wait_store(N_CHUNKS - 2, 0)
        wait_store(N_CHUNKS - 1, 1)

    return k(packed, indices)


def kernel(x, table):
    indices = x.reshape(N).astype(jnp.int32)
    packed = _sc_pack(table)
    return _sc_gather(packed, indices)


# speedup vs baseline: 1.0050x; 1.0050x over previous
import dataclasses
import functools

import jax
import jax.numpy as jnp
from jax import lax
from jax.experimental import pallas as pl
from jax.experimental.pallas import tpu as pltpu
from jax.experimental.pallas import tpu_sc as plsc

B = 16384
L = 20
EMB = 32
VOCAB = 1000000
PACK = 128 // EMB
N = B * L

NC = 2
NS = 16
NW = NC * NS
LK_PER_W = N // NW
B_PER_W = B // NW
CHUNK = 80
CBATCH = CHUNK // L
N_CHUNKS = LK_PER_W // CHUNK
LANES = 16


def _compiler_params():
    cp = pltpu.CompilerParams()
    if "needs_layout_passes" in pltpu.CompilerParams.__dataclass_fields__:
        cp = dataclasses.replace(cp, needs_layout_passes=False)
    return cp


PK_TC = 2000
SEG = VOCAB // PACK


def _tc_pack(table):
    def body(i0, i1, i2, i3, out_ref):
        for k, ref in enumerate((i0, i1, i2, i3)):
            out_ref[:, k * EMB:(k + 1) * EMB] = ref[...]

    nblk = SEG // PK_TC

    def mk_in(k):
        return pl.BlockSpec((PK_TC, EMB), lambda i, kk=k: (kk * nblk + i, 0))

    return pl.pallas_call(
        body,
        grid=(nblk,),
        in_specs=[mk_in(k) for k in range(PACK)],
        out_specs=pl.BlockSpec((PK_TC, PACK * EMB), lambda i: (i, 0)),
        out_shape=jax.ShapeDtypeStruct((SEG, PACK * EMB), jnp.float32),
        compiler_params=pltpu.CompilerParams(
            dimension_semantics=("arbitrary",),
        ),
    )(table, table, table, table)


def _sc_gather(packed, indices):
    mesh = plsc.VectorSubcoreMesh(core_axis_name="c", subcore_axis_name="s")

    @functools.partial(
        pl.kernel,
        mesh=mesh,
        compiler_params=_compiler_params(),
        out_type=jax.ShapeDtypeStruct((B, L, EMB), jnp.float32),
        scratch_types=[
            pltpu.VMEM((LK_PER_W,), jnp.int32),
            pltpu.VMEM((LK_PER_W,), jnp.int32),
            pltpu.VMEM((CHUNK, 128), jnp.float32),
            pltpu.VMEM((CHUNK, 128), jnp.float32),
            pltpu.VMEM((CHUNK, EMB), jnp.float32),
            pltpu.VMEM((CHUNK, EMB), jnp.float32),
            pltpu.SemaphoreType.DMA,
            pltpu.SemaphoreType.DMA,
            pltpu.SemaphoreType.DMA,
            pltpu.SemaphoreType.DMA,
        ],
    )
    def k(tbl_hbm, idx_hbm, out_hbm, idx_v, pidx_v, b0, b1, o0, o1,
          g0, g1, s0, s1):
        wid = lax.axis_index("s") * NC + lax.axis_index("c")
        lbase = wid * LK_PER_W
        bbase = wid * B_PER_W

        bufs = (b0, b1)
        outs = (o0, o1)
        gsems = (g0, g1)
        ssems = (s0, s1)

        pltpu.sync_copy(idx_hbm.at[pl.ds(lbase, LK_PER_W)], idx_v)

        @pl.loop(0, LK_PER_W, step=LANES)
        def _(i):
            v = idx_v[pl.ds(i, LANES)]
            kk = (
                (v >= SEG).astype(jnp.int32)
                + (v >= 2 * SEG).astype(jnp.int32)
                + (v >= 3 * SEG).astype(jnp.int32)
            )
            pidx_v[pl.ds(i, LANES)] = v - kk * SEG
            idx_v[pl.ds(i, LANES)] = kk * EMB

        def start_gather(c, p):
            pltpu.make_async_copy(
                tbl_hbm.at[pidx_v.at[pl.ds(c * CHUNK, CHUNK)]],
                bufs[p], gsems[p],
            ).start()

        def wait_gather(c, p):
            pltpu.make_async_copy(
                tbl_hbm.at[pidx_v.at[pl.ds(c * CHUNK, CHUNK)]],
                bufs[p], gsems[p],
            ).wait()

        def start_store(c, p):
            for bb in range(CBATCH):
                pltpu.make_async_copy(
                    outs[p].at[pl.ds(bb * L, L)],
                    out_hbm.at[bbase + c * CBATCH + bb],
                    ssems[p],
                ).start()

        def wait_store(c, p):
            for bb in range(CBATCH):
                pltpu.make_async_copy(
                    outs[p].at[pl.ds(bb * L, L)],
                    out_hbm.at[bbase + c * CBATCH + bb],
                    ssems[p],
                ).wait()

        def select(c, p):
            buf, o = bufs[p], outs[p]

            @pl.loop(0, CHUNK, step=LANES)
            def _(t):
                colb = idx_v[pl.ds(c * CHUNK + t, LANES)]
                for j in range(LANES):
                    q = colb[j]
                    r = t + j
                    o[r, pl.ds(0, LANES)] = buf[r, pl.ds(q, LANES)]
                    o[r, pl.ds(LANES, LANES)] = buf[r, pl.ds(q + LANES, LANES)]

        start_gather(0, 0)
        start_gather(1, 1)

        @pl.loop(0, N_CHUNKS // 2)
        def _(c2):
            for p in range(2):
                c = c2 * 2 + p
                wait_gather(c, p)

                @pl.when(c2 > 0)
                def _():
                    wait_store(c - 2, p)

                select(c, p)
                start_store(c, p)

                @pl.when(c + 2 < N_CHUNKS)
                def _():
                    start_gather(c + 2, p)

        wait_store(N_CHUNKS - 2, 0)
        wait_store(N_CHUNKS - 1, 1)

    return k(packed, indices)


def kernel(x, table):
    indices = x.reshape(N).astype(jnp.int32)
    packed = _tc_pack(table)
    return _sc_gather(packed, indices)


# speedup vs baseline: 1.0388x; 1.0337x over previous
import dataclasses
import functools

import jax
import jax.numpy as jnp
from jax import lax
from jax.experimental import pallas as pl
from jax.experimental.pallas import tpu as pltpu
from jax.experimental.pallas import tpu_sc as plsc

B = 16384
L = 20
EMB = 32
VOCAB = 1000000
PACK = 128 // EMB
N = B * L

NC = 2
NS = 16
NW = NC * NS
LK_PER_W = N // NW
B_PER_W = B // NW
CHUNK = 80
CBATCH = CHUNK // L
N_CHUNKS = LK_PER_W // CHUNK
LANES = 16


def _compiler_params():
    cp = pltpu.CompilerParams()
    if "needs_layout_passes" in pltpu.CompilerParams.__dataclass_fields__:
        cp = dataclasses.replace(cp, needs_layout_passes=False)
    return cp


PK_TC = 10000
SEG = VOCAB // PACK


def _tc_pack(table):
    def body(i0, i1, i2, i3, out_ref):
        for k, ref in enumerate((i0, i1, i2, i3)):
            out_ref[:, k * EMB:(k + 1) * EMB] = ref[...]

    nblk = SEG // PK_TC

    def mk_in(k):
        return pl.BlockSpec((PK_TC, EMB), lambda i, kk=k: (kk * nblk + i, 0))

    return pl.pallas_call(
        body,
        grid=(nblk,),
        in_specs=[mk_in(k) for k in range(PACK)],
        out_specs=pl.BlockSpec((PK_TC, PACK * EMB), lambda i: (i, 0)),
        out_shape=jax.ShapeDtypeStruct((SEG, PACK * EMB), jnp.float32),
        compiler_params=pltpu.CompilerParams(
            dimension_semantics=("parallel",),
        ),
    )(table, table, table, table)


def _sc_gather(packed, indices):
    mesh = plsc.VectorSubcoreMesh(core_axis_name="c", subcore_axis_name="s")

    @functools.partial(
        pl.kernel,
        mesh=mesh,
        compiler_params=_compiler_params(),
        out_type=jax.ShapeDtypeStruct((B, L, EMB), jnp.float32),
        scratch_types=[
            pltpu.VMEM((LK_PER_W,), jnp.int32),
            pltpu.VMEM((LK_PER_W,), jnp.int32),
            pltpu.VMEM((CHUNK, 128), jnp.float32),
            pltpu.VMEM((CHUNK, 128), jnp.float32),
            pltpu.VMEM((CHUNK, EMB), jnp.float32),
            pltpu.VMEM((CHUNK, EMB), jnp.float32),
            pltpu.SemaphoreType.DMA,
            pltpu.SemaphoreType.DMA,
            pltpu.SemaphoreType.DMA,
            pltpu.SemaphoreType.DMA,
        ],
    )
    def k(tbl_hbm, idx_hbm, out_hbm, idx_v, pidx_v, b0, b1, o0, o1,
          g0, g1, s0, s1):
        wid = lax.axis_index("s") * NC + lax.axis_index("c")
        lbase = wid * LK_PER_W
        bbase = wid * B_PER_W

        bufs = (b0, b1)
        outs = (o0, o1)
        gsems = (g0, g1)
        ssems = (s0, s1)

        pltpu.sync_copy(idx_hbm.at[pl.ds(lbase, LK_PER_W)], idx_v)

        @pl.loop(0, LK_PER_W, step=LANES)
        def _(i):
            v = idx_v[pl.ds(i, LANES)]
            kk = (
                (v >= SEG).astype(jnp.int32)
                + (v >= 2 * SEG).astype(jnp.int32)
                + (v >= 3 * SEG).astype(jnp.int32)
            )
            pidx_v[pl.ds(i, LANES)] = v - kk * SEG
            idx_v[pl.ds(i, LANES)] = kk * EMB

        def start_gather(c, p):
            pltpu.make_async_copy(
                tbl_hbm.at[pidx_v.at[pl.ds(c * CHUNK, CHUNK)]],
                bufs[p], gsems[p],
            ).start()

        def wait_gather(c, p):
            pltpu.make_async_copy(
                tbl_hbm.at[pidx_v.at[pl.ds(c * CHUNK, CHUNK)]],
                bufs[p], gsems[p],
            ).wait()

        def start_store(c, p):
            for bb in range(CBATCH):
                pltpu.make_async_copy(
                    outs[p].at[pl.ds(bb * L, L)],
                    out_hbm.at[bbase + c * CBATCH + bb],
                    ssems[p],
                ).start()

        def wait_store(c, p):
            for bb in range(CBATCH):
                pltpu.make_async_copy(
                    outs[p].at[pl.ds(bb * L, L)],
                    out_hbm.at[bbase + c * CBATCH + bb],
                    ssems[p],
                ).wait()

        def select(c, p):
            buf, o = bufs[p], outs[p]

            @pl.loop(0, CHUNK, step=LANES)
            def _(t):
                colb = idx_v[pl.ds(c * CHUNK + t, LANES)]
                for j in range(LANES):
                    q = colb[j]
                    r = t + j
                    o[r, pl.ds(0, LANES)] = buf[r, pl.ds(q, LANES)]
                    o[r, pl.ds(LANES, LANES)] = buf[r, pl.ds(q + LANES, LANES)]

        start_gather(0, 0)
        start_gather(1, 1)

        @pl.loop(0, N_CHUNKS // 2)
        def _(c2):
            for p in range(2):
                c = c2 * 2 + p
                wait_gather(c, p)

                @pl.when(c2 > 0)
                def _():
                    wait_store(c - 2, p)

                select(c, p)
                start_store(c, p)

                @pl.when(c + 2 < N_CHUNKS)
                def _():
                    start_gather(c + 2, p)

        wait_store(N_CHUNKS - 2, 0)
        wait_store(N_CHUNKS - 1, 1)

    return k(packed, indices)


def kernel(x, table):
    indices = x.reshape(N).astype(jnp.int32)
    packed = _tc_pack(table)
    return _sc_gather(packed, indices)


# speedup vs baseline: 1.0482x; 1.0090x over previous
import dataclasses
import functools

import jax
import jax.numpy as jnp
from jax import lax
from jax.experimental import pallas as pl
from jax.experimental.pallas import tpu as pltpu
from jax.experimental.pallas import tpu_sc as plsc

B = 16384
L = 20
EMB = 32
VOCAB = 1000000
PACK = 128 // EMB
N = B * L

NC = 2
NS = 16
NW = NC * NS
LK_PER_W = N // NW
B_PER_W = B // NW
CHUNK = 80
CBATCH = CHUNK // L
N_CHUNKS = LK_PER_W // CHUNK
LANES = 16


def _compiler_params():
    cp = pltpu.CompilerParams()
    if "needs_layout_passes" in pltpu.CompilerParams.__dataclass_fields__:
        cp = dataclasses.replace(cp, needs_layout_passes=False)
    return cp


def _sc_gather(packed, indices):
    mesh = plsc.VectorSubcoreMesh(core_axis_name="c", subcore_axis_name="s")

    @functools.partial(
        pl.kernel,
        mesh=mesh,
        compiler_params=_compiler_params(),
        out_type=jax.ShapeDtypeStruct((B, L, EMB), jnp.float32),
        scratch_types=[
            pltpu.VMEM((LK_PER_W,), jnp.int32),
            pltpu.VMEM((LK_PER_W,), jnp.int32),
            pltpu.VMEM((CHUNK, 128), jnp.float32),
            pltpu.VMEM((CHUNK, 128), jnp.float32),
            pltpu.VMEM((CHUNK, EMB), jnp.float32),
            pltpu.VMEM((CHUNK, EMB), jnp.float32),
            pltpu.SemaphoreType.DMA,
            pltpu.SemaphoreType.DMA,
            pltpu.SemaphoreType.DMA,
            pltpu.SemaphoreType.DMA,
        ],
    )
    def k(tbl_hbm, idx_hbm, out_hbm, idx_v, pidx_v, b0, b1, o0, o1,
          g0, g1, s0, s1):
        wid = lax.axis_index("s") * NC + lax.axis_index("c")
        lbase = wid * LK_PER_W
        bbase = wid * B_PER_W

        bufs = (b0, b1)
        outs = (o0, o1)
        gsems = (g0, g1)
        ssems = (s0, s1)

        pltpu.sync_copy(idx_hbm.at[pl.ds(lbase, LK_PER_W)], idx_v)

        @pl.loop(0, LK_PER_W, step=LANES)
        def _(i):
            pidx_v[pl.ds(i, LANES)] = idx_v[pl.ds(i, LANES)] >> 2

        def start_gather(c, p):
            pltpu.make_async_copy(
                tbl_hbm.at[pidx_v.at[pl.ds(c * CHUNK, CHUNK)]],
                bufs[p], gsems[p],
            ).start()

        def wait_gather(c, p):
            pltpu.make_async_copy(
                tbl_hbm.at[pidx_v.at[pl.ds(c * CHUNK, CHUNK)]],
                bufs[p], gsems[p],
            ).wait()

        def start_store(c, p):
            for bb in range(CBATCH):
                pltpu.make_async_copy(
                    outs[p].at[pl.ds(bb * L, L)],
                    out_hbm.at[bbase + c * CBATCH + bb],
                    ssems[p],
                ).start()

        def wait_store(c, p):
            for bb in range(CBATCH):
                pltpu.make_async_copy(
                    outs[p].at[pl.ds(bb * L, L)],
                    out_hbm.at[bbase + c * CBATCH + bb],
                    ssems[p],
                ).wait()

        def select(c, p):
            buf, o = bufs[p], outs[p]

            @pl.loop(0, CHUNK, step=LANES)
            def _(t):
                colb = (idx_v[pl.ds(c * CHUNK + t, LANES)] & 3) * EMB
                for j in range(LANES):
                    q = colb[j]
                    r = t + j
                    o[r, pl.ds(0, LANES)] = buf[r, pl.ds(q, LANES)]
                    o[r, pl.ds(LANES, LANES)] = buf[r, pl.ds(q + LANES, LANES)]

        start_gather(0, 0)
        start_gather(1, 1)

        @pl.loop(0, N_CHUNKS // 2)
        def _(c2):
            for p in range(2):
                c = c2 * 2 + p
                wait_gather(c, p)

                @pl.when(c2 > 0)
                def _():
                    wait_store(c - 2, p)

                select(c, p)
                start_store(c, p)

                @pl.when(c + 2 < N_CHUNKS)
                def _():
                    start_gather(c + 2, p)

        wait_store(N_CHUNKS - 2, 0)
        wait_store(N_CHUNKS - 1, 1)

    return k(packed, indices)


def kernel(x, table):
    indices = x.reshape(N).astype(jnp.int32)
    packed = table.reshape(VOCAB // PACK, PACK * EMB)
    return _sc_gather(packed, indices)
